# Initial kernel scaffold; baseline (speedup 1.0000x reference)
import jax
import jax.numpy as jnp
from jax.experimental import pallas as pl


def kernel(latent, condition, edge_index, W_z2h, b_z2h, W_c2h, b_c2h, W_h2h, b_h2h, W_out, b_out):
    raise NotImplementedError("write your pallas kernel here")


# speedup vs baseline: 10.2696x; 10.2696x over previous
import jax
import jax.numpy as jnp
from jax import lax
from jax.experimental import pallas as pl
from jax.experimental.pallas import tpu as pltpu
import jax.experimental.pallas.tpu_sc as plsc

NC = 2
NS = 16
CHUNK = 80
BS = 1024
NP = 10240
D = 128

_MESH = plsc.VectorSubcoreMesh(core_axis_name="c", subcore_axis_name="s")
_HIGH = lax.Precision.HIGHEST


def _dot(a, b):
    return jax.lax.dot(a, b, precision=_HIGH, preferred_element_type=jnp.float32)


def _deg_body(col_hbm, hist_hbm, hist_v, idx_v):
    c = lax.axis_index("c")
    s = lax.axis_index("s")
    wid = c * NS + s
    z16 = jnp.zeros((16,), jnp.float32)

    @pl.loop(0, NP // 16)
    def _(i):
        hist_v[pl.ds(i * 16, 16)] = z16

    e = col_hbm.shape[0]
    per_tile = e // (NC * NS)
    base0 = wid * per_tile
    ones16 = jnp.ones((16,), jnp.float32)

    @pl.loop(0, per_tile // CHUNK)
    def _(j):
        pltpu.sync_copy(col_hbm.at[pl.ds(base0 + j * CHUNK, CHUNK)], idx_v)
        for k in range(CHUNK // 16):
            iv = idx_v[pl.ds(k * 16, 16)]
            plsc.addupdate_scatter(hist_v, [iv], ones16)

    pltpu.sync_copy(hist_v, hist_hbm.at[wid])


def _deg_call(col):
    return pl.kernel(
        _deg_body,
        out_type=jax.ShapeDtypeStruct((NC * NS, NP), jnp.float32),
        mesh=_MESH,
        scratch_types=(
            pltpu.VMEM((NP,), jnp.float32),
            pltpu.VMEM((CHUNK,), jnp.int32),
        ),
        compiler_params=pltpu.CompilerParams(needs_layout_passes=False),
    )(col)


def _accum_edges(u_hbm, row_hbm, col_hbm, table, idx_r, idx_c, rows_v, sem,
                 rbase0, cbase0, nchunks):
    @pl.loop(0, nchunks)
    def _(j):
        pltpu.sync_copy(row_hbm.at[pl.ds(rbase0 + j * CHUNK, CHUNK)], idx_r)
        pltpu.sync_copy(col_hbm.at[pl.ds(cbase0 + j * CHUNK, CHUNK)], idx_c)
        pltpu.async_copy(u_hbm.at[idx_r], rows_v, sem).wait()
        pltpu.sync_copy(rows_v, table.at[idx_c], add=True)


def _agg_dual_body(u2_hbm, row2_hbm, col_hbm, zeros_hbm, s2_hbm,
                   table, idx_r, idx_c, rows_v, sem):
    c = lax.axis_index("c")
    s = lax.axis_index("s")
    rpt = NP // NS
    pltpu.sync_copy(zeros_hbm.at[pl.ds(s * rpt, rpt)],
                    table.at[pl.ds(s * rpt, rpt)])
    plsc.subcore_barrier()
    e = col_hbm.shape[0]
    per_tile = e // NS
    _accum_edges(u2_hbm, row2_hbm, col_hbm, table, idx_r, idx_c, rows_v, sem,
                 c * e + s * per_tile, s * per_tile, per_tile // CHUNK)
    plsc.subcore_barrier()
    pltpu.sync_copy(table.at[pl.ds(s * rpt, rpt)],
                    s2_hbm.at[pl.ds(c * NP + s * rpt, rpt)])


def _agg_split_body(u_hbm, row_hbm, col_hbm, zeros_hbm, s2_hbm,
                    table, idx_r, idx_c, rows_v, sem):
    c = lax.axis_index("c")
    s = lax.axis_index("s")
    rpt = NP // NS
    pltpu.sync_copy(zeros_hbm.at[pl.ds(s * rpt, rpt)],
                    table.at[pl.ds(s * rpt, rpt)])
    plsc.subcore_barrier()
    e = col_hbm.shape[0]
    per_tile = e // (NC * NS)
    base0 = (c * NS + s) * per_tile
    _accum_edges(u_hbm, row_hbm, col_hbm, table, idx_r, idx_c, rows_v, sem,
                 base0, base0, per_tile // CHUNK)
    plsc.subcore_barrier()
    pltpu.sync_copy(table.at[pl.ds(s * rpt, rpt)],
                    s2_hbm.at[pl.ds(c * NP + s * rpt, rpt)])


def _agg_scratch():
    return (
        pltpu.VMEM_SHARED((NP, D), jnp.float32),
        pltpu.VMEM((CHUNK,), jnp.int32),
        pltpu.VMEM((CHUNK,), jnp.int32),
        pltpu.VMEM((CHUNK, D), jnp.float32),
        pltpu.SemaphoreType.DMA,
    )


_S2 = jax.ShapeDtypeStruct((NC * NP, D), jnp.float32)


def _agg_dual_call(u2, row2, col, zeros_nd):
    return pl.kernel(_agg_dual_body, out_type=_S2, mesh=_MESH,
                     scratch_types=_agg_scratch())(u2, row2, col, zeros_nd)


def _agg_split_call(u, row, col, zeros_nd):
    return pl.kernel(_agg_split_body, out_type=_S2, mesh=_MESH,
                     scratch_types=_agg_scratch())(u, row, col, zeros_nd)


def _row_spec(width=D):
    return pl.BlockSpec((BS, width), lambda i: (i, 0))


def _hi_spec():
    return pl.BlockSpec((BS, D), lambda i: (NP // BS + i, 0))


def _full_spec(shape):
    return pl.BlockSpec(shape, lambda i: tuple(0 for _ in shape))


def _dinv_body(hist, dinv_o):
    deg = jnp.sum(hist[...], axis=0) + 1.0
    dinv_o[...] = jax.lax.rsqrt(deg)[:, None]


def _dinv_call(hist):
    hb = 1024
    return pl.pallas_call(
        _dinv_body,
        grid=(NP // hb,),
        in_specs=[pl.BlockSpec((NC * NS, hb), lambda i: (0, i))],
        out_specs=pl.BlockSpec((hb, 1), lambda i: (i, 0)),
        out_shape=jax.ShapeDtypeStruct((NP, 1), jnp.float32),
    )(hist)


def _prep_body(dinv, lat, cond, wz, wc, uz_o, uc_o):
    d = dinv[...]
    uz_o[...] = d * _dot(lat[...], wz[...])
    uc_o[...] = d * _dot(cond[...], wc[...])


def _prep_call(dinv, latent, condition, wz, wc):
    return pl.pallas_call(
        _prep_body,
        grid=(NP // BS,),
        in_specs=[pl.BlockSpec((BS, 1), lambda i: (i, 0)),
                  _row_spec(), _row_spec(),
                  _full_spec((D, D)), _full_spec((D, D))],
        out_specs=[_row_spec(), _row_spec()],
        out_shape=[jax.ShapeDtypeStruct((NP, D), jnp.float32),
                   jax.ShapeDtypeStruct((NP, D), jnp.float32)],
    )(dinv, latent, condition, wz, wc)


def _mid_body(sz, sc_, uz, uc, dinv, bz, bc, wh, uh_o):
    d = dinv[...]
    z = jnp.tanh(d * (sz[...] + uz[...]) + bz[...])
    c2 = jnp.tanh(d * (sc_[...] + uc[...]) + bc[...])
    uh_o[...] = d * (_dot(z, wh[0:D, :]) + _dot(c2, wh[D:2 * D, :]))


def _mid_call(s2, uz, uc, dinv, bz, bc, wh):
    return pl.pallas_call(
        _mid_body,
        grid=(NP // BS,),
        in_specs=[_row_spec(), _hi_spec(), _row_spec(), _row_spec(),
                  pl.BlockSpec((BS, 1), lambda i: (i, 0)),
                  _full_spec((1, D)), _full_spec((1, D)),
                  _full_spec((2 * D, D))],
        out_specs=_row_spec(),
        out_shape=jax.ShapeDtypeStruct((NP, D), jnp.float32),
    )(s2, s2, uz, uc, dinv, bz, bc, wh)


def _outprep_body(s0, s1, uh, dinv, bh, wo, uo_o):
    d = dinv[...]
    h = jnp.tanh(d * (s0[...] + s1[...] + uh[...]) + bh[...])
    uo_o[...] = d * _dot(h, wo[...])


def _outprep_call(s2, uh, dinv, bh, wo):
    return pl.pallas_call(
        _outprep_body,
        grid=(NP // BS,),
        in_specs=[_row_spec(), _hi_spec(), _row_spec(),
                  pl.BlockSpec((BS, 1), lambda i: (i, 0)),
                  _full_spec((1, D)), _full_spec((D, D))],
        out_specs=_row_spec(),
        out_shape=jax.ShapeDtypeStruct((NP, D), jnp.float32),
    )(s2, s2, uh, dinv, bh, wo)


def _final_body(s0, s1, uo, dinv, bo, out_o):
    out_o[...] = dinv[...] * (s0[...] + s1[...] + uo[...]) + bo[...]


def _final_call(s2, uo, dinv, bo):
    return pl.pallas_call(
        _final_body,
        grid=(NP // BS,),
        in_specs=[_row_spec(), _hi_spec(), _row_spec(),
                  pl.BlockSpec((BS, 1), lambda i: (i, 0)),
                  _full_spec((1, D))],
        out_specs=_row_spec(),
        out_shape=jax.ShapeDtypeStruct((NP, D), jnp.float32),
    )(s2, s2, uo, dinv, bo)


def kernel(latent, condition, edge_index, W_z2h, b_z2h, W_c2h, b_c2h,
           W_h2h, b_h2h, W_out, b_out):
    n = latent.shape[0]
    row = edge_index[0]
    col = edge_index[1]
    row2 = jnp.concatenate([row, row + NP])
    zeros_nd = jnp.zeros((NP, D), jnp.float32)
    lat_p = jnp.pad(latent, ((0, NP - n), (0, 0)))
    cond_p = jnp.pad(condition, ((0, NP - n), (0, 0)))

    hist = _deg_call(col)
    dinv = _dinv_call(hist)
    u_z, u_c = _prep_call(dinv, lat_p, cond_p, W_z2h, W_c2h)
    u2 = jnp.concatenate([u_z, u_c], axis=0)
    s2a = _agg_dual_call(u2, row2, col, zeros_nd)
    u_h = _mid_call(s2a, u_z, u_c, dinv, b_z2h.reshape(1, D),
                    b_c2h.reshape(1, D), W_h2h)
    s2h = _agg_split_call(u_h, row, col, zeros_nd)
    u_o = _outprep_call(s2h, u_h, dinv, b_h2h.reshape(1, D), W_out)
    s2o = _agg_split_call(u_o, row, col, zeros_nd)
    return _final_call(s2o, u_o, dinv, b_out.reshape(1, D))[:n]


# speedup vs baseline: 24.4078x; 2.3767x over previous
import jax
import jax.numpy as jnp
from jax import lax
from jax.experimental import pallas as pl
from jax.experimental.pallas import tpu as pltpu
import jax.experimental.pallas.tpu_sc as plsc

NC = 2
NS = 16
CHUNK = 80
BS = 1024
NP = 10240
D = 128

_MESH = plsc.VectorSubcoreMesh(core_axis_name="c", subcore_axis_name="s")
_HIGH = lax.Precision.HIGHEST


def _dot(a, b):
    return jax.lax.dot(a, b, precision=_HIGH, preferred_element_type=jnp.float32)


def _deg_body(col_hbm, hist_hbm, hist_v, idx_v):
    c = lax.axis_index("c")
    s = lax.axis_index("s")
    wid = c * NS + s
    z16 = jnp.zeros((16,), jnp.float32)

    @pl.loop(0, NP // 16)
    def _(i):
        hist_v[pl.ds(i * 16, 16)] = z16

    e = col_hbm.shape[0]
    per_tile = e // (NC * NS)
    pltpu.sync_copy(col_hbm.at[pl.ds(wid * per_tile, per_tile)], idx_v)
    ones16 = jnp.ones((16,), jnp.float32)

    @pl.loop(0, per_tile // 16)
    def _(k):
        iv = idx_v[pl.ds(k * 16, 16)]
        plsc.addupdate_scatter(hist_v, [iv], ones16)

    pltpu.sync_copy(hist_v, hist_hbm.at[wid])


def _deg_call(col):
    e = col.shape[0]
    return pl.kernel(
        _deg_body,
        out_type=jax.ShapeDtypeStruct((NC * NS, NP), jnp.float32),
        mesh=_MESH,
        scratch_types=(
            pltpu.VMEM((NP,), jnp.float32),
            pltpu.VMEM((e // (NC * NS),), jnp.int32),
        ),
        compiler_params=pltpu.CompilerParams(needs_layout_passes=False),
    )(col)


PHE = 10000


def _accum_edges(u_hbm, row_hbm, col_hbm, table, idx_r, idx_c,
                 buf0, buf1, g0, g1, rbase0, cbase0, n_edges):
    for ph in range(n_edges // PHE):
        _accum_phase(u_hbm, row_hbm, col_hbm, table, idx_r, idx_c,
                     buf0, buf1, g0, g1,
                     rbase0 + ph * PHE, cbase0 + ph * PHE, PHE)


def _accum_phase(u_hbm, row_hbm, col_hbm, table, idx_r, idx_c,
                 buf0, buf1, g0, g1, rbase0, cbase0, n_edges):
    nch = n_edges // CHUNK
    pltpu.sync_copy(row_hbm.at[pl.ds(rbase0, n_edges)], idx_r)
    pltpu.sync_copy(col_hbm.at[pl.ds(cbase0, n_edges)], idx_c)

    def gstart(j, buf, sem):
        pltpu.async_copy(u_hbm.at[idx_r.at[pl.ds(j * CHUNK, CHUNK)]], buf, sem)

    def gwait(buf, sem):
        pltpu.make_async_copy(u_hbm.at[pl.ds(0, CHUNK)], buf, sem).wait()

    def scatter(j, buf):
        pltpu.sync_copy(buf, table.at[idx_c.at[pl.ds(j * CHUNK, CHUNK)]],
                        add=True)

    gstart(0, buf0, g0)

    @pl.loop(0, nch // 2)
    def _(p):
        j = 2 * p
        gstart(j + 1, buf1, g1)
        gwait(buf0, g0)
        scatter(j, buf0)

        @pl.when(j + 2 < nch)
        def _():
            gstart(j + 2, buf0, g0)

        gwait(buf1, g1)
        scatter(j + 1, buf1)

    if nch % 2:
        gwait(buf0, g0)
        scatter(nch - 1, buf0)


def _agg_dual_body(u2_hbm, row2_hbm, col_hbm, zeros_hbm, s2_hbm,
                   table, idx_r, idx_c, buf0, buf1, g0, g1):
    c = lax.axis_index("c")
    s = lax.axis_index("s")
    rpt = NP // NS
    pltpu.sync_copy(zeros_hbm.at[pl.ds(s * rpt, rpt)],
                    table.at[pl.ds(s * rpt, rpt)])
    plsc.subcore_barrier()
    e = col_hbm.shape[0]
    per_tile = e // NS
    _accum_edges(u2_hbm, row2_hbm, col_hbm, table, idx_r, idx_c,
                 buf0, buf1, g0, g1,
                 c * e + s * per_tile, s * per_tile, per_tile)
    plsc.subcore_barrier()
    pltpu.sync_copy(table.at[pl.ds(s * rpt, rpt)],
                    s2_hbm.at[pl.ds(c * NP + s * rpt, rpt)])


def _agg_split_body(u_hbm, row_hbm, col_hbm, zeros_hbm, s2_hbm,
                    table, idx_r, idx_c, buf0, buf1, g0, g1):
    c = lax.axis_index("c")
    s = lax.axis_index("s")
    rpt = NP // NS
    pltpu.sync_copy(zeros_hbm.at[pl.ds(s * rpt, rpt)],
                    table.at[pl.ds(s * rpt, rpt)])
    plsc.subcore_barrier()
    e = col_hbm.shape[0]
    per_tile = e // (NC * NS)
    base0 = (c * NS + s) * per_tile
    _accum_edges(u_hbm, row_hbm, col_hbm, table, idx_r, idx_c,
                 buf0, buf1, g0, g1, base0, base0, per_tile)
    plsc.subcore_barrier()
    pltpu.sync_copy(table.at[pl.ds(s * rpt, rpt)],
                    s2_hbm.at[pl.ds(c * NP + s * rpt, rpt)])


def _agg_scratch():
    return (
        pltpu.VMEM_SHARED((NP, D), jnp.float32),
        pltpu.VMEM((PHE,), jnp.int32),
        pltpu.VMEM((PHE,), jnp.int32),
        pltpu.VMEM((CHUNK, D), jnp.float32),
        pltpu.VMEM((CHUNK, D), jnp.float32),
        pltpu.SemaphoreType.DMA,
        pltpu.SemaphoreType.DMA,
    )


_S2 = jax.ShapeDtypeStruct((NC * NP, D), jnp.float32)


def _agg_dual_call(u2, row2, col, zeros_nd):
    return pl.kernel(_agg_dual_body, out_type=_S2, mesh=_MESH,
                     scratch_types=_agg_scratch())(u2, row2, col, zeros_nd)


def _agg_split_call(u, row, col, zeros_nd):
    return pl.kernel(_agg_split_body, out_type=_S2, mesh=_MESH,
                     scratch_types=_agg_scratch())(u, row, col, zeros_nd)


def _row_spec(width=D):
    return pl.BlockSpec((BS, width), lambda i: (i, 0))


def _hi_spec():
    return pl.BlockSpec((BS, D), lambda i: (NP // BS + i, 0))


def _full_spec(shape):
    return pl.BlockSpec(shape, lambda i: tuple(0 for _ in shape))


def _dinv_body(hist, dinv_o):
    deg = jnp.sum(hist[...], axis=0) + 1.0
    dinv_o[...] = jax.lax.rsqrt(deg)[:, None]


def _dinv_call(hist):
    hb = 1024
    return pl.pallas_call(
        _dinv_body,
        grid=(NP // hb,),
        in_specs=[pl.BlockSpec((NC * NS, hb), lambda i: (0, i))],
        out_specs=pl.BlockSpec((hb, 1), lambda i: (i, 0)),
        out_shape=jax.ShapeDtypeStruct((NP, 1), jnp.float32),
    )(hist)


def _prep_body(dinv, lat, cond, wz, wc, uz_o, uc_o):
    d = dinv[...]
    uz_o[...] = d * _dot(lat[...], wz[...])
    uc_o[...] = d * _dot(cond[...], wc[...])


def _prep_call(dinv, latent, condition, wz, wc):
    return pl.pallas_call(
        _prep_body,
        grid=(NP // BS,),
        in_specs=[pl.BlockSpec((BS, 1), lambda i: (i, 0)),
                  _row_spec(), _row_spec(),
                  _full_spec((D, D)), _full_spec((D, D))],
        out_specs=[_row_spec(), _row_spec()],
        out_shape=[jax.ShapeDtypeStruct((NP, D), jnp.float32),
                   jax.ShapeDtypeStruct((NP, D), jnp.float32)],
    )(dinv, latent, condition, wz, wc)


def _mid_body(sz, sc_, uz, uc, dinv, bz, bc, wh, uh_o):
    d = dinv[...]
    z = jnp.tanh(d * (sz[...] + uz[...]) + bz[...])
    c2 = jnp.tanh(d * (sc_[...] + uc[...]) + bc[...])
    uh_o[...] = d * (_dot(z, wh[0:D, :]) + _dot(c2, wh[D:2 * D, :]))


def _mid_call(s2, uz, uc, dinv, bz, bc, wh):
    return pl.pallas_call(
        _mid_body,
        grid=(NP // BS,),
        in_specs=[_row_spec(), _hi_spec(), _row_spec(), _row_spec(),
                  pl.BlockSpec((BS, 1), lambda i: (i, 0)),
                  _full_spec((1, D)), _full_spec((1, D)),
                  _full_spec((2 * D, D))],
        out_specs=_row_spec(),
        out_shape=jax.ShapeDtypeStruct((NP, D), jnp.float32),
    )(s2, s2, uz, uc, dinv, bz, bc, wh)


def _outprep_body(s0, s1, uh, dinv, bh, wo, uo_o):
    d = dinv[...]
    h = jnp.tanh(d * (s0[...] + s1[...] + uh[...]) + bh[...])
    uo_o[...] = d * _dot(h, wo[...])


def _outprep_call(s2, uh, dinv, bh, wo):
    return pl.pallas_call(
        _outprep_body,
        grid=(NP // BS,),
        in_specs=[_row_spec(), _hi_spec(), _row_spec(),
                  pl.BlockSpec((BS, 1), lambda i: (i, 0)),
                  _full_spec((1, D)), _full_spec((D, D))],
        out_specs=_row_spec(),
        out_shape=jax.ShapeDtypeStruct((NP, D), jnp.float32),
    )(s2, s2, uh, dinv, bh, wo)


def _final_body(s0, s1, uo, dinv, bo, out_o):
    out_o[...] = dinv[...] * (s0[...] + s1[...] + uo[...]) + bo[...]


def _final_call(s2, uo, dinv, bo):
    return pl.pallas_call(
        _final_body,
        grid=(NP // BS,),
        in_specs=[_row_spec(), _hi_spec(), _row_spec(),
                  pl.BlockSpec((BS, 1), lambda i: (i, 0)),
                  _full_spec((1, D))],
        out_specs=_row_spec(),
        out_shape=jax.ShapeDtypeStruct((NP, D), jnp.float32),
    )(s2, s2, uo, dinv, bo)


def kernel(latent, condition, edge_index, W_z2h, b_z2h, W_c2h, b_c2h,
           W_h2h, b_h2h, W_out, b_out):
    n = latent.shape[0]
    row = edge_index[0]
    col = edge_index[1]
    row2 = jnp.concatenate([row, row + NP])
    zeros_nd = jnp.zeros((NP, D), jnp.float32)
    lat_p = jnp.pad(latent, ((0, NP - n), (0, 0)))
    cond_p = jnp.pad(condition, ((0, NP - n), (0, 0)))

    hist = _deg_call(col)
    dinv = _dinv_call(hist)
    u_z, u_c = _prep_call(dinv, lat_p, cond_p, W_z2h, W_c2h)
    u2 = jnp.concatenate([u_z, u_c], axis=0)
    s2a = _agg_dual_call(u2, row2, col, zeros_nd)
    u_h = _mid_call(s2a, u_z, u_c, dinv, b_z2h.reshape(1, D),
                    b_c2h.reshape(1, D), W_h2h)
    s2h = _agg_split_call(u_h, row, col, zeros_nd)
    u_o = _outprep_call(s2h, u_h, dinv, b_h2h.reshape(1, D), W_out)
    s2o = _agg_split_call(u_o, row, col, zeros_nd)
    return _final_call(s2o, u_o, dinv, b_out.reshape(1, D))[:n]


# speedup vs baseline: 25.0527x; 1.0264x over previous
import jax
import jax.numpy as jnp
from jax import lax
from jax.experimental import pallas as pl
from jax.experimental.pallas import tpu as pltpu
import jax.experimental.pallas.tpu_sc as plsc

NC = 2
NS = 16
CHUNK = 80
BS = 1024
NP = 10240
D = 128

_MESH = plsc.VectorSubcoreMesh(core_axis_name="c", subcore_axis_name="s")
_HIGH = lax.Precision.HIGHEST


def _dot(a, b):
    return jax.lax.dot(a, b, precision=_HIGH, preferred_element_type=jnp.float32)


def _deg_body(col_hbm, hist_hbm, hist_v, idx_v):
    c = lax.axis_index("c")
    s = lax.axis_index("s")
    wid = c * NS + s
    z16 = jnp.zeros((16,), jnp.float32)

    @pl.loop(0, NP // 16)
    def _(i):
        hist_v[pl.ds(i * 16, 16)] = z16

    e = col_hbm.shape[0]
    per_tile = e // (NC * NS)
    pltpu.sync_copy(col_hbm.at[pl.ds(wid * per_tile, per_tile)], idx_v)
    ones16 = jnp.ones((16,), jnp.float32)

    @pl.loop(0, per_tile // 16)
    def _(k):
        iv = idx_v[pl.ds(k * 16, 16)]
        plsc.addupdate_scatter(hist_v, [iv], ones16)

    pltpu.sync_copy(hist_v, hist_hbm.at[wid])


def _deg_call(col):
    e = col.shape[0]
    return pl.kernel(
        _deg_body,
        out_type=jax.ShapeDtypeStruct((NC * NS, NP), jnp.float32),
        mesh=_MESH,
        scratch_types=(
            pltpu.VMEM((NP,), jnp.float32),
            pltpu.VMEM((e // (NC * NS),), jnp.int32),
        ),
        compiler_params=pltpu.CompilerParams(needs_layout_passes=False),
    )(col)


PHE = 10000


def _accum_edges(u_hbm, row_hbm, col_hbm, table, idx_r, idx_c,
                 buf0, buf1, g0, g1, rbase0, cbase0, n_edges):
    for ph in range(n_edges // PHE):
        _accum_phase(u_hbm, row_hbm, col_hbm, table, idx_r, idx_c,
                     buf0, buf1, g0, g1,
                     rbase0 + ph * PHE, cbase0 + ph * PHE, PHE)


def _accum_phase(u_hbm, row_hbm, col_hbm, table, idx_r, idx_c,
                 buf0, buf1, g0, g1, rbase0, cbase0, n_edges):
    nch = n_edges // CHUNK
    pltpu.sync_copy(row_hbm.at[pl.ds(rbase0, n_edges)], idx_r)
    pltpu.sync_copy(col_hbm.at[pl.ds(cbase0, n_edges)], idx_c)

    def gstart(j, buf, sem):
        pltpu.async_copy(u_hbm.at[idx_r.at[pl.ds(j * CHUNK, CHUNK)]], buf, sem)

    def gwait(buf, sem):
        pltpu.make_async_copy(u_hbm.at[pl.ds(0, CHUNK)], buf, sem).wait()

    def scatter(j, buf):
        pltpu.sync_copy(buf, table.at[idx_c.at[pl.ds(j * CHUNK, CHUNK)]],
                        add=True)

    gstart(0, buf0, g0)

    @pl.loop(0, nch // 2)
    def _(p):
        j = 2 * p
        gstart(j + 1, buf1, g1)
        gwait(buf0, g0)
        scatter(j, buf0)

        @pl.when(j + 2 < nch)
        def _():
            gstart(j + 2, buf0, g0)

        gwait(buf1, g1)
        scatter(j + 1, buf1)

    if nch % 2:
        gwait(buf0, g0)
        scatter(nch - 1, buf0)


def _agg_dual_body(u2_hbm, row2_hbm, col_hbm, zeros_hbm, s2_hbm,
                   table, idx_r, idx_c, buf0, buf1, g0, g1):
    c = lax.axis_index("c")
    s = lax.axis_index("s")
    rpt = NP // NS
    pltpu.sync_copy(zeros_hbm.at[pl.ds(s * rpt, rpt)],
                    table.at[pl.ds(s * rpt, rpt)])
    plsc.subcore_barrier()
    e = col_hbm.shape[0]
    per_tile = e // NS
    _accum_edges(u2_hbm, row2_hbm, col_hbm, table, idx_r, idx_c,
                 buf0, buf1, g0, g1,
                 c * e + s * per_tile, s * per_tile, per_tile)
    plsc.subcore_barrier()
    pltpu.sync_copy(table.at[pl.ds(s * rpt, rpt)],
                    s2_hbm.at[pl.ds(c * NP + s * rpt, rpt)])


def _agg_split_body(u_hbm, row_hbm, col_hbm, zeros_hbm, s2_hbm,
                    table, idx_r, idx_c, buf0, buf1, g0, g1):
    c = lax.axis_index("c")
    s = lax.axis_index("s")
    rpt = NP // NS
    pltpu.sync_copy(zeros_hbm.at[pl.ds(s * rpt, rpt)],
                    table.at[pl.ds(s * rpt, rpt)])
    plsc.subcore_barrier()
    e = col_hbm.shape[0]
    per_tile = e // (NC * NS)
    base0 = (c * NS + s) * per_tile
    _accum_edges(u_hbm, row_hbm, col_hbm, table, idx_r, idx_c,
                 buf0, buf1, g0, g1, base0, base0, per_tile)
    plsc.subcore_barrier()
    pltpu.sync_copy(table.at[pl.ds(s * rpt, rpt)],
                    s2_hbm.at[pl.ds(c * NP + s * rpt, rpt)])


def _agg_scratch():
    return (
        pltpu.VMEM_SHARED((NP, D), jnp.float32),
        pltpu.VMEM((PHE,), jnp.int32),
        pltpu.VMEM((PHE,), jnp.int32),
        pltpu.VMEM((CHUNK, D), jnp.float32),
        pltpu.VMEM((CHUNK, D), jnp.float32),
        pltpu.SemaphoreType.DMA,
        pltpu.SemaphoreType.DMA,
    )


_S2 = jax.ShapeDtypeStruct((NC * NP, D), jnp.float32)


def _agg_dual_call(u2, row2, col, zeros_nd):
    return pl.kernel(_agg_dual_body, out_type=_S2, mesh=_MESH,
                     scratch_types=_agg_scratch())(u2, row2, col, zeros_nd)


def _agg_split_call(u, row, col, zeros_nd):
    return pl.kernel(_agg_split_body, out_type=_S2, mesh=_MESH,
                     scratch_types=_agg_scratch())(u, row, col, zeros_nd)


def _row_spec(width=D):
    return pl.BlockSpec((BS, width), lambda i: (i, 0))


def _hi_spec():
    return pl.BlockSpec((BS, D), lambda i: (NP // BS + i, 0))


def _full_spec(shape):
    return pl.BlockSpec(shape, lambda i: tuple(0 for _ in shape))


def _prep_body(hist, lat, cond, wz, wc, dinv_o, uz_o, uc_o):
    deg = jnp.sum(hist[...], axis=0) + 1.0
    d = jax.lax.rsqrt(deg)[:, None]
    dinv_o[...] = d
    uz_o[...] = d * _dot(lat[...], wz[...])
    uc_o[...] = d * _dot(cond[...], wc[...])


def _prep_call(hist, latent, condition, wz, wc):
    n = latent.shape[0]
    return pl.pallas_call(
        _prep_body,
        grid=(NP // BS,),
        in_specs=[pl.BlockSpec((NC * NS, BS), lambda i: (0, i)),
                  _row_spec(), _row_spec(),
                  _full_spec((D, D)), _full_spec((D, D))],
        out_specs=[pl.BlockSpec((BS, 1), lambda i: (i, 0)),
                   _row_spec(), _row_spec()],
        out_shape=[jax.ShapeDtypeStruct((n, 1), jnp.float32),
                   jax.ShapeDtypeStruct((n, D), jnp.float32),
                   jax.ShapeDtypeStruct((n, D), jnp.float32)],
    )(hist, latent, condition, wz, wc)


def _mid_body(sz, sc_, uz, uc, dinv, bz, bc, wh, uh_o):
    d = dinv[...]
    z = jnp.tanh(d * (sz[...] + uz[...]) + bz[...])
    c2 = jnp.tanh(d * (sc_[...] + uc[...]) + bc[...])
    uh_o[...] = d * (_dot(z, wh[0:D, :]) + _dot(c2, wh[D:2 * D, :]))


def _mid_call(s2, uz, uc, dinv, bz, bc, wh):
    n = uz.shape[0]
    return pl.pallas_call(
        _mid_body,
        grid=(NP // BS,),
        in_specs=[_row_spec(), _hi_spec(), _row_spec(), _row_spec(),
                  pl.BlockSpec((BS, 1), lambda i: (i, 0)),
                  _full_spec((1, D)), _full_spec((1, D)),
                  _full_spec((2 * D, D))],
        out_specs=_row_spec(),
        out_shape=jax.ShapeDtypeStruct((uz.shape[0], D), jnp.float32),
    )(s2, s2, uz, uc, dinv, bz, bc, wh)


def _outprep_body(s0, s1, uh, dinv, bh, wo, uo_o):
    d = dinv[...]
    h = jnp.tanh(d * (s0[...] + s1[...] + uh[...]) + bh[...])
    uo_o[...] = d * _dot(h, wo[...])


def _outprep_call(s2, uh, dinv, bh, wo):
    return pl.pallas_call(
        _outprep_body,
        grid=(NP // BS,),
        in_specs=[_row_spec(), _hi_spec(), _row_spec(),
                  pl.BlockSpec((BS, 1), lambda i: (i, 0)),
                  _full_spec((1, D)), _full_spec((D, D))],
        out_specs=_row_spec(),
        out_shape=jax.ShapeDtypeStruct((uh.shape[0], D), jnp.float32),
    )(s2, s2, uh, dinv, bh, wo)


def _final_body(s0, s1, uo, dinv, bo, out_o):
    out_o[...] = dinv[...] * (s0[...] + s1[...] + uo[...]) + bo[...]


def _final_call(s2, uo, dinv, bo):
    return pl.pallas_call(
        _final_body,
        grid=(NP // BS,),
        in_specs=[_row_spec(), _hi_spec(), _row_spec(),
                  pl.BlockSpec((BS, 1), lambda i: (i, 0)),
                  _full_spec((1, D))],
        out_specs=_row_spec(),
        out_shape=jax.ShapeDtypeStruct((uo.shape[0], D), jnp.float32),
    )(s2, s2, uo, dinv, bo)


def kernel(latent, condition, edge_index, W_z2h, b_z2h, W_c2h, b_c2h,
           W_h2h, b_h2h, W_out, b_out):
    n = latent.shape[0]
    row = edge_index[0]
    col = edge_index[1]
    row2 = jnp.concatenate([row, row + n])
    zeros_nd = jnp.zeros((NP, D), jnp.float32)

    hist = _deg_call(col)
    dinv, u_z, u_c = _prep_call(hist, latent, condition, W_z2h, W_c2h)
    u2 = jnp.concatenate([u_z, u_c], axis=0)
    s2a = _agg_dual_call(u2, row2, col, zeros_nd)
    u_h = _mid_call(s2a, u_z, u_c, dinv, b_z2h.reshape(1, D),
                    b_c2h.reshape(1, D), W_h2h)
    s2h = _agg_split_call(u_h, row, col, zeros_nd)
    u_o = _outprep_call(s2h, u_h, dinv, b_h2h.reshape(1, D), W_out)
    s2o = _agg_split_call(u_o, row, col, zeros_nd)
    return _final_call(s2o, u_o, dinv, b_out.reshape(1, D))
